# baseline (speedup 1.0000x reference)
import functools
import jax
import jax.numpy as jnp
from jax import lax
from jax.experimental import pallas as pl
from jax.experimental.pallas import tpu as pltpu
from jax.experimental.pallas import tpu_sc as plsc

_TEMPERATURE = 0.7
_TOP_P = 0.9
_SAMPLE_KEY = 42
_BIG = 2**30
_ROWS = 8

_VP = 100352
_NROWS = 128
_NW = 32
_RPW = _NROWS // _NW
_CHUNK = 2048
_NCHUNK = _VP // _CHUNK
_VPC = _CHUNK // 16


def _radix_pass_kernel(p, keys_hbm, out_hbm, bufA, stage_k, stage_i, hist,
                       base, sem):
    cid = lax.axis_index("c")
    sid = lax.axis_index("s")
    wid = sid * 2 + cid
    lane = lax.broadcasted_iota(jnp.int32, (16,), 0)

    def do_row(t, _):
        row = wid * _RPW + t
        rbase = row * _VP
        pltpu.sync_copy(keys_hbm.at[pl.ds(rbase, _VP)], bufA)

        def zero_body(i, _):
            hist[pl.ds(i * 16, 16)] = jnp.zeros((16,), jnp.int32)
            return 0
        lax.fori_loop(0, 4096 // 16, zero_body, 0)

        def hist_body(i, _):
            k = bufA[pl.ds(i * 16, 16)]
            nk = k ^ jnp.uint32(0xFFFFFFFF)
            d = (lax.shift_right_logical(nk, jnp.uint32(p * 8))
                 & jnp.uint32(0xFF)).astype(jnp.int32)
            plsc.addupdate_scatter(hist, [d * 16 + lane],
                                   jnp.ones((16,), jnp.int32))
            return 0
        lax.fori_loop(0, _VP // 16, hist_body, 0)

        def scan_dv(dv, carry):
            cnt = jnp.zeros((16,), jnp.int32)
            for r in range(16):
                cnt = cnt + plsc.load_gather(
                    hist, [(dv * 16 + lane) * 16 + r])
            cs = plsc.cumsum(cnt)
            base[pl.ds(dv * 16, 16)] = cs - cnt + carry
            return carry + jnp.sum(cnt, axis=0)
        lax.fori_loop(0, 16, scan_dv, jnp.int32(0))

        def do_chunk(c, _):
            def do_vreg(ci, _):
                k = bufA[pl.ds((c * _VPC + ci) * 16, 16)]
                nk = k ^ jnp.uint32(0xFFFFFFFF)
                d = (lax.shift_right_logical(nk, jnp.uint32(p * 8))
                     & jnp.uint32(0xFF)).astype(jnp.int32)
                occ, is_last = plsc.scan_count(d)
                b = plsc.load_gather(base, [d])
                off = b + occ - 1 + rbase
                plsc.addupdate_scatter(base, [d], occ, mask=is_last)
                stage_k[pl.ds(ci * 16, 16)] = k
                stage_i[pl.ds(ci * 16, 16)] = off
                return 0
            lax.fori_loop(0, _VPC, do_vreg, 0)
            pltpu.async_copy(stage_k, out_hbm.at[stage_i], sem).wait()
            return 0
        lax.fori_loop(0, _NCHUNK, do_chunk, 0)
        return 0

    lax.fori_loop(0, _RPW, do_row, 0)


def _make_pass(p):
    mesh = plsc.VectorSubcoreMesh(core_axis_name="c", subcore_axis_name="s")
    return pl.kernel(
        functools.partial(_radix_pass_kernel, p),
        out_type=jax.ShapeDtypeStruct((_NROWS * _VP,), jnp.uint32),
        mesh=mesh,
        compiler_params=pltpu.CompilerParams(needs_layout_passes=False),
        scratch_types=[
            pltpu.VMEM((_VP,), jnp.uint32),
            pltpu.VMEM((_CHUNK,), jnp.uint32),
            pltpu.VMEM((_CHUNK,), jnp.int32),
            pltpu.VMEM((4096,), jnp.int32),
            pltpu.VMEM((256,), jnp.int32),
            pltpu.SemaphoreType.DMA,
        ],
    )


def _sc_sort_desc(probs):
    v = probs.shape[-1]
    keys = lax.bitcast_convert_type(probs, jnp.uint32)
    keys = jnp.pad(keys, ((0, 0), (0, _VP - v)))
    x = keys.reshape(-1)
    for p in range(4):
        x = _make_pass(p)(x)
    out = x.reshape(_NROWS, _VP)[:, :v]
    return lax.bitcast_convert_type(out, jnp.float32)


def _cumsum_last(x):
    n = x.shape[-1]
    k = 1
    while k < n:
        shifted = jnp.concatenate(
            [jnp.zeros(x.shape[:-1] + (k,), x.dtype), x[:, : n - k]], axis=-1
        )
        x = x + shifted
        k *= 2
    return x


def _sample_body(probs_ref, ps_ref, g_ref, out_ref):
    ps = ps_ref[...]
    cs = _cumsum_last(ps)
    kept = jnp.where((cs - ps) > _TOP_P, 0.0, ps)
    s = jnp.sum(kept, axis=-1, keepdims=True)
    q = kept / s
    v = jnp.log(q + 1e-20) + g_ref[...]
    iota = jax.lax.broadcasted_iota(jnp.int32, v.shape, 1)
    maxv = jnp.max(v, axis=-1, keepdims=True)
    jstar = jnp.min(jnp.where(v == maxv, iota, _BIG), axis=-1, keepdims=True)
    vstar = jnp.sum(jnp.where(iota == jstar, ps, 0.0), axis=-1, keepdims=True)
    pr = probs_ref[...]
    cnt_gt = jnp.sum((pr > vstar).astype(jnp.int32), axis=-1, keepdims=True)
    m = jstar - cnt_gt
    eq = pr == vstar
    eqcs = _cumsum_last(eq.astype(jnp.int32))
    tok = jnp.min(jnp.where(eq & (eqcs == m + 1), iota, _BIG), axis=-1)
    out_ref[...] = jnp.broadcast_to(tok[:, None], out_ref.shape).astype(jnp.int32)


def kernel(logits):
    b, v = logits.shape
    probs = jax.nn.softmax(logits / _TEMPERATURE, axis=-1)
    probs_sort = _sc_sort_desc(probs)
    g = jax.random.gumbel(jax.random.key(_SAMPLE_KEY), probs.shape, probs.dtype)
    row_spec = pl.BlockSpec((_ROWS, v), lambda i: (i, 0))
    out = pl.pallas_call(
        _sample_body,
        grid=(b // _ROWS,),
        in_specs=[row_spec, row_spec, row_spec],
        out_specs=pl.BlockSpec((_ROWS, 128), lambda i: (i, 0)),
        out_shape=jax.ShapeDtypeStruct((b, 128), jnp.int32),
    )(probs, probs_sort, g)
    return out[:, 0]
